# Initial kernel scaffold; baseline (speedup 1.0000x reference)
import jax
import jax.numpy as jnp
from jax.experimental import pallas as pl


def kernel(x, W, b):
    raise NotImplementedError("write your pallas kernel here")


# speedup vs baseline: 6.1483x; 6.1483x over previous
import jax
import jax.numpy as jnp
from jax.experimental import pallas as pl

_NUM_BLOCKS = 64
_BT = 1024


def _gate_body(x_ref, w_ref, b_ref, o_ref):
    logits = jax.lax.dot_general(
        x_ref[...], w_ref[...],
        (((1,), (1,)), ((), ())),
        preferred_element_type=jnp.float32,
    ) + b_ref[...]
    ids = jax.lax.broadcasted_iota(jnp.int32, logits.shape, 1)
    big = jnp.int32(_NUM_BLOCKS)
    m1 = jnp.max(logits, axis=1, keepdims=True)
    idx1 = jnp.min(jnp.where(logits == m1, ids, big), axis=1, keepdims=True)
    hit1 = ids == idx1
    masked = jnp.where(hit1, -jnp.inf, logits)
    m2 = jnp.max(masked, axis=1, keepdims=True)
    idx2 = jnp.min(jnp.where(masked == m2, ids, big), axis=1, keepdims=True)
    o_ref[...] = (hit1 | (ids == idx2)).astype(jnp.float32)


def kernel(x, W, b):
    n = x.shape[0]
    return pl.pallas_call(
        _gate_body,
        grid=(n // _BT,),
        in_specs=[
            pl.BlockSpec((_BT, x.shape[1]), lambda i: (i, 0)),
            pl.BlockSpec((_NUM_BLOCKS, x.shape[1]), lambda i: (0, 0)),
            pl.BlockSpec((1, _NUM_BLOCKS), lambda i: (0, 0)),
        ],
        out_specs=pl.BlockSpec((_BT, _NUM_BLOCKS), lambda i: (i, 0)),
        out_shape=jax.ShapeDtypeStruct((n, _NUM_BLOCKS), jnp.float32),
    )(x, W, b[None, :])


# speedup vs baseline: 6.4105x; 1.0427x over previous
import jax
import jax.numpy as jnp
from jax.experimental import pallas as pl

_NUM_BLOCKS = 64
_BT = 1024


def _gate_body(x_ref, w_ref, b_ref, o_ref):
    logits = jax.lax.dot_general(
        x_ref[...], w_ref[...],
        (((1,), (1,)), ((), ())),
        preferred_element_type=jnp.float32,
    ) + b_ref[...]
    m1 = jnp.max(logits, axis=1, keepdims=True)
    c1 = logits == m1
    c1f = c1.astype(jnp.float32)
    masked = jnp.where(c1, -jnp.inf, logits)
    m2 = jnp.max(masked, axis=1, keepdims=True)
    c2 = masked == m2
    fe = jax.lax.broadcasted_iota(jnp.int32, (_NUM_BLOCKS, _NUM_BLOCKS), 0)
    ee = jax.lax.broadcasted_iota(jnp.int32, (_NUM_BLOCKS, _NUM_BLOCKS), 1)
    tri = (fe <= ee).astype(jnp.float32)
    a = c1f + c2.astype(jnp.float32) * (1.0 / 64.0)
    p = jax.lax.dot_general(a, tri, (((1,), (0,)), ((), ())),
                            preferred_element_type=jnp.float32)
    p2 = jnp.floor(p)
    p1 = (p - p2) * 64.0
    n1 = jnp.sum(c1f, axis=1, keepdims=True)
    sel = (c1 & (p2 <= 2.0)) | (c2 & (n1 == 1.0) & (p1 <= 1.0))
    o_ref[...] = sel.astype(jnp.float32)


def kernel(x, W, b):
    n = x.shape[0]
    return pl.pallas_call(
        _gate_body,
        grid=(n // _BT,),
        in_specs=[
            pl.BlockSpec((_BT, x.shape[1]), lambda i: (i, 0)),
            pl.BlockSpec((_NUM_BLOCKS, x.shape[1]), lambda i: (0, 0)),
            pl.BlockSpec((1, _NUM_BLOCKS), lambda i: (0, 0)),
        ],
        out_specs=pl.BlockSpec((_BT, _NUM_BLOCKS), lambda i: (i, 0)),
        out_shape=jax.ShapeDtypeStruct((n, _NUM_BLOCKS), jnp.float32),
    )(x, W, b[None, :])


# speedup vs baseline: 7.5455x; 1.1771x over previous
import jax
import jax.numpy as jnp
from jax.experimental import pallas as pl

_NUM_BLOCKS = 64
_BT = 2048


def _gate_body(x_ref, w_ref, b_ref, o_ref):
    logits = jax.lax.dot_general(
        x_ref[...], w_ref[...],
        (((1,), (1,)), ((), ())),
        preferred_element_type=jnp.float32,
    ) + b_ref[...]
    m1 = jnp.max(logits, axis=1, keepdims=True)
    c1 = logits == m1
    c1f = c1.astype(jnp.float32)
    masked = jnp.where(c1, -jnp.inf, logits)
    m2 = jnp.max(masked, axis=1, keepdims=True)
    c2 = masked == m2
    fe = jax.lax.broadcasted_iota(jnp.int32, (_NUM_BLOCKS, _NUM_BLOCKS), 0)
    ee = jax.lax.broadcasted_iota(jnp.int32, (_NUM_BLOCKS, _NUM_BLOCKS), 1)
    tri = (fe <= ee).astype(jnp.float32)
    a = c1f + c2.astype(jnp.float32) * (1.0 / 64.0)
    p = jax.lax.dot_general(a, tri, (((1,), (0,)), ((), ())),
                            preferred_element_type=jnp.float32)
    p2 = jnp.floor(p)
    p1 = (p - p2) * 64.0
    n1 = jnp.sum(c1f, axis=1, keepdims=True)
    sel = (c1 & (p2 <= 2.0)) | (c2 & (n1 == 1.0) & (p1 <= 1.0))
    o_ref[...] = sel.astype(jnp.float32)


def kernel(x, W, b):
    n = x.shape[0]
    return pl.pallas_call(
        _gate_body,
        grid=(n // _BT,),
        in_specs=[
            pl.BlockSpec((_BT, x.shape[1]), lambda i: (i, 0)),
            pl.BlockSpec((_NUM_BLOCKS, x.shape[1]), lambda i: (0, 0)),
            pl.BlockSpec((1, _NUM_BLOCKS), lambda i: (0, 0)),
        ],
        out_specs=pl.BlockSpec((_BT, _NUM_BLOCKS), lambda i: (i, 0)),
        out_shape=jax.ShapeDtypeStruct((n, _NUM_BLOCKS), jnp.float32),
    )(x, W, b[None, :])


# speedup vs baseline: 8.1154x; 1.0755x over previous
import jax
import jax.numpy as jnp
from jax.experimental import pallas as pl

_NUM_BLOCKS = 64
_BT = 4096


def _gate_body(x_ref, w_ref, b_ref, o_ref):
    logits = jax.lax.dot_general(
        x_ref[...], w_ref[...],
        (((1,), (1,)), ((), ())),
        preferred_element_type=jnp.float32,
    ) + b_ref[...]
    m1 = jnp.max(logits, axis=1, keepdims=True)
    c1 = logits == m1
    c1f = c1.astype(jnp.float32)
    masked = jnp.where(c1, -jnp.inf, logits)
    m2 = jnp.max(masked, axis=1, keepdims=True)
    c2 = masked == m2
    fe = jax.lax.broadcasted_iota(jnp.int32, (_NUM_BLOCKS, _NUM_BLOCKS), 0)
    ee = jax.lax.broadcasted_iota(jnp.int32, (_NUM_BLOCKS, _NUM_BLOCKS), 1)
    tri = (fe <= ee).astype(jnp.float32)
    a = c1f + c2.astype(jnp.float32) * (1.0 / 64.0)
    p = jax.lax.dot_general(a, tri, (((1,), (0,)), ((), ())),
                            preferred_element_type=jnp.float32)
    p2 = jnp.floor(p)
    p1 = (p - p2) * 64.0
    n1 = jnp.sum(c1f, axis=1, keepdims=True)
    sel = (c1 & (p2 <= 2.0)) | (c2 & (n1 == 1.0) & (p1 <= 1.0))
    o_ref[...] = sel.astype(jnp.float32)


def kernel(x, W, b):
    n = x.shape[0]
    return pl.pallas_call(
        _gate_body,
        grid=(n // _BT,),
        in_specs=[
            pl.BlockSpec((_BT, x.shape[1]), lambda i: (i, 0)),
            pl.BlockSpec((_NUM_BLOCKS, x.shape[1]), lambda i: (0, 0)),
            pl.BlockSpec((1, _NUM_BLOCKS), lambda i: (0, 0)),
        ],
        out_specs=pl.BlockSpec((_BT, _NUM_BLOCKS), lambda i: (i, 0)),
        out_shape=jax.ShapeDtypeStruct((n, _NUM_BLOCKS), jnp.float32),
    )(x, W, b[None, :])
